# Initial kernel scaffold; baseline (speedup 1.0000x reference)
import jax
import jax.numpy as jnp
from jax.experimental import pallas as pl


def kernel(atom_features, atom_coords, edge_index, pair_indices, pair_features, pair_coords, W_embed, b_embed, gW1, gb1, gW2, gb2, conv_W0, conv_b0, conv_W1, conv_b1, conv_W2, conv_b2, bn_g0, bn_b0, bn_g1, bn_b1, bn_g2, bn_b2, pW0, pb0, pW1, pb1, pW2, pb2):
    raise NotImplementedError("write your pallas kernel here")


# speedup vs baseline: 6.4328x; 6.4328x over previous
import functools

import jax
import jax.numpy as jnp
from jax import lax
from jax.experimental import pallas as pl
from jax.experimental.pallas import tpu as pltpu
from jax.experimental.pallas import tpu_sc as plsc

_N = 10000
_E = 320000
_P = 100000
_P2 = 102400
_H = 256
_HH = 128
_EPS = 1e-5
_BNS = float(1.0 / (1.0 + _EPS) ** 0.5)

_NC = 2
_NS = 16
_K = 80
_EPT = _E // _NS
_SLAB = 624
_PK = 128
_PPW = _P2 // (_NC * _NS)


def _sc_mesh():
    return plsc.VectorSubcoreMesh(core_axis_name="c", subcore_axis_name="s")


def _per_tile_slab(s, do):
    @pl.when(s < _NS - 1)
    def _():
        do(s * _SLAB, _SLAB)

    @pl.when(s == _NS - 1)
    def _():
        do((_NS - 1) * _SLAB, _N - (_NS - 1) * _SLAB)


def _sc_degree(dst, zeros_nh, ones_kh):
    @functools.partial(
        pl.kernel,
        mesh=_sc_mesh(),
        out_type=jax.ShapeDtypeStruct((_NC, _N, _HH), jnp.float32),
        scratch_types=[
            pltpu.VMEM((_K,), jnp.int32),
            pltpu.VMEM((_K, _HH), jnp.float32),
            pltpu.VMEM_SHARED((_N, _HH), jnp.float32),
        ],
    )
    def k(dst_hbm, zero_hbm, ones_hbm, out_hbm, didx, ones_v, acc):
        c = lax.axis_index("c")
        s = lax.axis_index("s")
        _per_tile_slab(s, lambda b, n: pltpu.sync_copy(
            zero_hbm.at[pl.ds(b, n)], acc.at[pl.ds(b, n)]))
        pltpu.sync_copy(ones_hbm, ones_v)
        plsc.subcore_barrier()
        ebase = c * (_E // _NC) + s * (_E // (_NC * _NS))
        nchunks = _E // (_NC * _NS * _K)

        def body(i, carry):
            b = ebase + i * _K
            pltpu.sync_copy(dst_hbm.at[pl.ds(b, _K)], didx)
            pltpu.sync_copy(ones_v, acc.at[didx], add=True)
            return carry

        lax.fori_loop(0, nchunks, body, 0, unroll=False)
        plsc.subcore_barrier()
        _per_tile_slab(s, lambda b, n: pltpu.sync_copy(
            acc.at[pl.ds(b, n)], out_hbm.at[c, pl.ds(b, n)]))

    return k(dst, zeros_nh, ones_kh)


def _sc_conv_scatter(xs_flat, src_cat, dst):
    @functools.partial(
        pl.kernel,
        mesh=_sc_mesh(),
        out_type=jax.ShapeDtypeStruct((_NC, _N, _HH), jnp.float32),
        scratch_types=[
            pltpu.VMEM((_K,), jnp.int32),
            pltpu.VMEM((_K,), jnp.int32),
            pltpu.VMEM((_K, _HH), jnp.float32),
            pltpu.VMEM_SHARED((_N, _HH), jnp.float32),
            pltpu.SemaphoreType.DMA,
        ],
    )
    def k(xs_hbm, src_hbm, dst_hbm, out_hbm, sidx, didx, rows, acc, sem):
        c = lax.axis_index("c")
        s = lax.axis_index("s")
        _per_tile_slab(s, lambda b, n: pltpu.sync_copy(
            xs_hbm.at[pl.ds(c * _N + b, n)], acc.at[pl.ds(b, n)]))
        plsc.subcore_barrier()
        ebase = s * _EPT
        nchunks = _EPT // _K

        def body(i, carry):
            b = ebase + i * _K
            pltpu.sync_copy(src_hbm.at[pl.ds(c * _E + b, _K)], sidx)
            pltpu.async_copy(xs_hbm.at[sidx], rows, sem).wait()
            pltpu.sync_copy(dst_hbm.at[pl.ds(b, _K)], didx)
            pltpu.sync_copy(rows, acc.at[didx], add=True)
            return carry

        lax.fori_loop(0, nchunks, body, 0, unroll=False)
        plsc.subcore_barrier()
        _per_tile_slab(s, lambda b, n: pltpu.sync_copy(
            acc.at[pl.ds(b, n)], out_hbm.at[c, pl.ds(b, n)]))

    return k(xs_flat, src_cat, dst)


def _sc_pair_gather(table, idx0, idx1):
    @functools.partial(
        pl.kernel,
        mesh=_sc_mesh(),
        out_type=(jax.ShapeDtypeStruct((_P2, _H), jnp.float32),
                  jax.ShapeDtypeStruct((_P2, _H), jnp.float32)),
        scratch_types=[
            pltpu.VMEM((_PK,), jnp.int32),
            pltpu.VMEM((_PK, _H), jnp.float32),
            pltpu.SemaphoreType.DMA,
        ],
    )
    def k(tab_hbm, i0_hbm, i1_hbm, a0_hbm, a1_hbm, idx, rows, sem):
        c = lax.axis_index("c")
        s = lax.axis_index("s")
        w = s * _NC + c
        base = w * _PPW
        nchunks = _PPW // _PK

        def body(i, carry):
            b = base + i * _PK
            pltpu.sync_copy(i0_hbm.at[pl.ds(b, _PK)], idx)
            pltpu.async_copy(tab_hbm.at[idx], rows, sem).wait()
            pltpu.sync_copy(rows, a0_hbm.at[pl.ds(b, _PK)])
            pltpu.sync_copy(i1_hbm.at[pl.ds(b, _PK)], idx)
            pltpu.async_copy(tab_hbm.at[idx], rows, sem).wait()
            pltpu.sync_copy(rows, a1_hbm.at[pl.ds(b, _PK)])
            return carry

        lax.fori_loop(0, nchunks, body, 0, unroll=False)

    return k(table, idx0, idx1)


_TB = 1000


def _dis_block(deg2):
    deg = deg2[0, :, 0:1] + deg2[1, :, 0:1] + 1.0
    return lax.rsqrt(deg)


def _c1_body(af, coords, deg2, we, be, gw1, gb1, gw2, gb2, w0, out):
    xe = jnp.dot(af[...], we[...], preferred_element_type=jnp.float32) + be[...]
    gh = jnp.maximum(
        jnp.dot(coords[...], gw1[...], preferred_element_type=jnp.float32) + gb1[...],
        0.0)
    gh = jnp.dot(gh, gw2[...], preferred_element_type=jnp.float32) + gb2[...]
    z = (jnp.dot(xe, w0[0:_H, :], preferred_element_type=jnp.float32)
         + jnp.dot(gh, w0[_H:, :], preferred_element_type=jnp.float32))
    xs = z * _dis_block(deg2[...])
    out[0, :, :] = xs[:, 0:_HH]
    out[1, :, :] = xs[:, _HH:]


def _tc_conv1(af, coords, deg2, we, be, gw1, gb1, gw2, gb2, w0):
    grid = _N // _TB
    return pl.pallas_call(
        _c1_body,
        grid=(grid,),
        in_specs=[
            pl.BlockSpec((_TB, 128), lambda i: (i, 0)),
            pl.BlockSpec((_TB, 3), lambda i: (i, 0)),
            pl.BlockSpec((_NC, _TB, _HH), lambda i: (0, i, 0)),
            pl.BlockSpec((128, _H), lambda i: (0, 0)),
            pl.BlockSpec((1, _H), lambda i: (0, 0)),
            pl.BlockSpec((3, 64), lambda i: (0, 0)),
            pl.BlockSpec((1, 64), lambda i: (0, 0)),
            pl.BlockSpec((64, 64), lambda i: (0, 0)),
            pl.BlockSpec((1, 64), lambda i: (0, 0)),
            pl.BlockSpec((_H + 64, _H), lambda i: (0, 0)),
        ],
        out_specs=pl.BlockSpec((_NC, _TB, _HH), lambda i: (0, i, 0)),
        out_shape=jax.ShapeDtypeStruct((_NC, _N, _HH), jnp.float32),
    )(af, coords, deg2, we, be, gw1, gb1, gw2, gb2, w0)


def _cmid_body(msg, deg2, b, gm, bt, w, out):
    dis = _dis_block(deg2[...])
    m = jnp.concatenate([msg[0, :, :], msg[1, :, :]], axis=1)
    x = jnp.maximum((m * dis + b[...]) * _BNS * gm[...] + bt[...], 0.0)
    xs = jnp.dot(x, w[...], preferred_element_type=jnp.float32) * dis
    out[0, :, :] = xs[:, 0:_HH]
    out[1, :, :] = xs[:, _HH:]


def _tc_conv_mid(msg, deg2, b, gm, bt, w):
    grid = _N // _TB
    return pl.pallas_call(
        _cmid_body,
        grid=(grid,),
        in_specs=[
            pl.BlockSpec((_NC, _TB, _HH), lambda i: (0, i, 0)),
            pl.BlockSpec((_NC, _TB, _HH), lambda i: (0, i, 0)),
            pl.BlockSpec((1, _H), lambda i: (0, 0)),
            pl.BlockSpec((1, _H), lambda i: (0, 0)),
            pl.BlockSpec((1, _H), lambda i: (0, 0)),
            pl.BlockSpec((_H, _H), lambda i: (0, 0)),
        ],
        out_specs=pl.BlockSpec((_NC, _TB, _HH), lambda i: (0, i, 0)),
        out_shape=jax.ShapeDtypeStruct((_NC, _N, _HH), jnp.float32),
    )(msg, deg2, b, gm, bt, w)


def _cfin_body(msg, deg2, b, gm, bt, out):
    dis = _dis_block(deg2[...])
    m = jnp.concatenate([msg[0, :, :], msg[1, :, :]], axis=1)
    out[...] = jnp.maximum((m * dis + b[...]) * _BNS * gm[...] + bt[...], 0.0)


def _tc_conv_fin(msg, deg2, b, gm, bt):
    grid = _N // _TB
    return pl.pallas_call(
        _cfin_body,
        grid=(grid,),
        in_specs=[
            pl.BlockSpec((_NC, _TB, _HH), lambda i: (0, i, 0)),
            pl.BlockSpec((_NC, _TB, _HH), lambda i: (0, i, 0)),
            pl.BlockSpec((1, _H), lambda i: (0, 0)),
            pl.BlockSpec((1, _H), lambda i: (0, 0)),
            pl.BlockSpec((1, _H), lambda i: (0, 0)),
        ],
        out_specs=pl.BlockSpec((_TB, _H), lambda i: (i, 0)),
        out_shape=jax.ShapeDtypeStruct((_N, _H), jnp.float32),
    )(msg, deg2, b, gm, bt)


_PB = 1024


def _pmlp_body(a0, a1, pfc, wa0, wa1, wf, b0, w1, b1, w2, b2, out):
    h = (jnp.dot(a0[...], wa0[...], preferred_element_type=jnp.float32)
         + jnp.dot(a1[...], wa1[...], preferred_element_type=jnp.float32)
         + jnp.dot(pfc[...], wf[...], preferred_element_type=jnp.float32)
         + b0[...])
    h = jnp.maximum(h, 0.0)
    h = jnp.maximum(jnp.dot(h, w1[...], preferred_element_type=jnp.float32) + b1[...], 0.0)
    out[...] = jnp.dot(h, w2[...], preferred_element_type=jnp.float32) + b2[...]


def _tc_pair_mlp(a0, a1, pfc, wa0, wa1, wf, b0, w1, b1, w2, b2):
    grid = _P2 // _PB
    h2 = 2 * _H
    return pl.pallas_call(
        _pmlp_body,
        grid=(grid,),
        in_specs=[
            pl.BlockSpec((_PB, _H), lambda i: (i, 0)),
            pl.BlockSpec((_PB, _H), lambda i: (i, 0)),
            pl.BlockSpec((_PB, 22), lambda i: (i, 0)),
            pl.BlockSpec((_H, h2), lambda i: (0, 0)),
            pl.BlockSpec((_H, h2), lambda i: (0, 0)),
            pl.BlockSpec((22, h2), lambda i: (0, 0)),
            pl.BlockSpec((1, h2), lambda i: (0, 0)),
            pl.BlockSpec((h2, _H), lambda i: (0, 0)),
            pl.BlockSpec((1, _H), lambda i: (0, 0)),
            pl.BlockSpec((_H, 1), lambda i: (0, 0)),
            pl.BlockSpec((1, 1), lambda i: (0, 0)),
        ],
        out_specs=pl.BlockSpec((_PB, 1), lambda i: (i, 0)),
        out_shape=jax.ShapeDtypeStruct((_P2, 1), jnp.float32),
    )(a0, a1, pfc, wa0, wa1, wf, b0, w1, b1, w2, b2)


def kernel(atom_features, atom_coords, edge_index, pair_indices, pair_features,
           pair_coords, W_embed, b_embed, gW1, gb1, gW2, gb2,
           conv_W0, conv_b0, conv_W1, conv_b1, conv_W2, conv_b2,
           bn_g0, bn_b0, bn_g1, bn_b1, bn_g2, bn_b2,
           pW0, pb0, pW1, pb1, pW2, pb2):
    src = edge_index[0].astype(jnp.int32)
    dst = edge_index[1].astype(jnp.int32)
    src_cat = jnp.concatenate([src, src + _N])

    npad = _P2 - _P
    padidx = (jnp.arange(npad, dtype=jnp.int32) * 37) % _N
    idx0 = jnp.concatenate([pair_indices[:, 0].astype(jnp.int32), padidx])
    idx1 = jnp.concatenate([pair_indices[:, 1].astype(jnp.int32), padidx])
    pfc = jnp.concatenate(
        [pair_features, pair_coords.reshape(_P, 6)], axis=1)
    pfc = jnp.concatenate(
        [pfc, jnp.zeros((npad, 22), jnp.float32)], axis=0)

    zeros_nh = jnp.zeros((_N, _HH), jnp.float32)
    ones_kh = jnp.ones((_K, _HH), jnp.float32)
    deg2 = _sc_degree(dst, zeros_nh, ones_kh)

    r = lambda v: v.reshape(1, -1)
    xs = _tc_conv1(atom_features, atom_coords, deg2, W_embed, r(b_embed),
                   gW1, r(gb1), gW2, r(gb2), conv_W0)
    msg = _sc_conv_scatter(xs.reshape(_NC * _N, _HH), src_cat, dst)
    xs = _tc_conv_mid(msg, deg2, r(conv_b0), r(bn_g0), r(bn_b0), conv_W1)
    msg = _sc_conv_scatter(xs.reshape(_NC * _N, _HH), src_cat, dst)
    xs = _tc_conv_mid(msg, deg2, r(conv_b1), r(bn_g1), r(bn_b1), conv_W2)
    msg = _sc_conv_scatter(xs.reshape(_NC * _N, _HH), src_cat, dst)
    x3 = _tc_conv_fin(msg, deg2, r(conv_b2), r(bn_g2), r(bn_b2))

    a0g, a1g = _sc_pair_gather(x3, idx0, idx1)
    out = _tc_pair_mlp(a0g, a1g, pfc, pW0[0:_H], pW0[_H:2 * _H],
                       pW0[2 * _H:], r(pb0), pW1, r(pb1), pW2, r(pb2))
    return out[:_P]


# speedup vs baseline: 13.4955x; 2.0979x over previous
import functools

import jax
import jax.numpy as jnp
from jax import lax
from jax.experimental import pallas as pl
from jax.experimental.pallas import tpu as pltpu
from jax.experimental.pallas import tpu_sc as plsc

_N = 10000
_E = 320000
_P = 100000
_P2 = 102400
_H = 256
_HH = 128
_EPS = 1e-5
_BNS = float(1.0 / (1.0 + _EPS) ** 0.5)

_NC = 2
_NS = 16
_SLAB = 624
_PK = 128
_PPW = _P2 // (_NC * _NS)
_PCH = _PPW // _PK
_NTRASH = 16
_E2 = 327680
_ECH = _E2 // (_NS * _PK)
_DCH = _E2 // (_NC * _NS * _PK)
_IBLK = 8
_NBLK = _ECH // _IBLK


def _sc_mesh():
    return plsc.VectorSubcoreMesh(core_axis_name="c", subcore_axis_name="s")


def _per_tile_slab(s, do):
    @pl.when(s < _NS - 1)
    def _():
        do(s * _SLAB, _SLAB)

    @pl.when(s == _NS - 1)
    def _():
        do((_NS - 1) * _SLAB, _N - (_NS - 1) * _SLAB)


def _sc_degree(dst2d, zeros_nh, ones_kh):
    @functools.partial(
        pl.kernel,
        mesh=_sc_mesh(),
        out_type=jax.ShapeDtypeStruct((_NC, _N, _HH), jnp.float32),
        scratch_types=[
            pltpu.VMEM((_DCH, _PK), jnp.int32),
            pltpu.VMEM((_PK, _HH), jnp.float32),
            pltpu.VMEM_SHARED((_N + _NTRASH, _HH), jnp.float32),
            pltpu.SemaphoreType.DMA,
        ],
    )
    def k(dst_hbm, zero_hbm, ones_hbm, out_hbm, didx, ones_v, acc, sem):
        c = lax.axis_index("c")
        s = lax.axis_index("s")
        w = s * _NC + c
        pltpu.sync_copy(dst_hbm.at[pl.ds(w * _DCH, _DCH)], didx)
        pltpu.sync_copy(ones_hbm, ones_v)
        _per_tile_slab(s, lambda b, n: pltpu.sync_copy(
            zero_hbm.at[pl.ds(b, n)], acc.at[pl.ds(b, n)]))
        plsc.subcore_barrier()

        def body(j, carry):
            for b in range(8):
                pltpu.async_copy(ones_v, acc.at[didx.at[j * 8 + b]], sem,
                                 add=True)
            for b in range(8):
                pltpu.make_async_copy(ones_v, acc.at[didx.at[0]], sem).wait()
            return carry

        lax.fori_loop(0, _DCH // 8, body, 0, unroll=False)
        plsc.subcore_barrier()
        _per_tile_slab(s, lambda b, n: pltpu.sync_copy(
            acc.at[pl.ds(b, n)], out_hbm.at[c, pl.ds(b, n)]))

    return k(dst2d, zeros_nh, ones_kh)


def _sc_conv_scatter(xs_flat, src2d, dst2d):
    @functools.partial(
        pl.kernel,
        mesh=_sc_mesh(),
        out_type=jax.ShapeDtypeStruct((_NC, _N, _HH), jnp.float32),
        scratch_types=[
            pltpu.VMEM((2 * _IBLK, _PK), jnp.int32),
            pltpu.VMEM((2 * _IBLK, _PK), jnp.int32),
            pltpu.VMEM((2, _PK, _HH), jnp.float32),
            pltpu.VMEM_SHARED((_N + _NTRASH, _HH), jnp.float32),
            pltpu.SemaphoreType.DMA,
            pltpu.SemaphoreType.DMA,
            pltpu.SemaphoreType.DMA,
            pltpu.SemaphoreType.DMA,
            pltpu.SemaphoreType.DMA,
            pltpu.SemaphoreType.DMA,
        ],
    )
    def k(xs_hbm, src_hbm, dst_hbm, out_hbm, sidx, didx, rows, acc,
          g0, g1, s0, s1, i0, i1):
        c = lax.axis_index("c")
        s = lax.axis_index("s")
        sbase = (c * _NS + s) * _ECH
        dbase = s * _ECH
        gsem = (g0, g1)
        ssem = (s0, s1)
        isem = (i0, i1)

        def fire_iblk(blk, slot):
            pltpu.async_copy(src_hbm.at[pl.ds(sbase + blk * _IBLK, _IBLK)],
                             sidx.at[pl.ds(slot * _IBLK, _IBLK)], isem[slot])
            pltpu.async_copy(dst_hbm.at[pl.ds(dbase + blk * _IBLK, _IBLK)],
                             didx.at[pl.ds(slot * _IBLK, _IBLK)], isem[slot])

        def wait_iblk(slot):
            for _ in range(2):
                pltpu.make_async_copy(
                    src_hbm.at[pl.ds(0, _IBLK)],
                    sidx.at[pl.ds(0, _IBLK)], isem[slot]).wait()

        def idxrow(ref, i):
            slot = lax.rem(i // _IBLK, 2)
            return ref.at[slot * _IBLK + lax.rem(i, _IBLK)]

        fire_iblk(0, 0)
        fire_iblk(1, 1)
        _per_tile_slab(s, lambda b, n: pltpu.sync_copy(
            xs_hbm.at[pl.ds(c * _N + b, n)], acc.at[pl.ds(b, n)]))
        plsc.subcore_barrier()
        wait_iblk(0)
        pltpu.async_copy(xs_hbm.at[sidx.at[0]], rows.at[0], g0)

        def body(t, carry):
            for b in (0, 1):
                i = t * 2 + b
                nb = 1 - b
                if b == 0:
                    pltpu.async_copy(xs_hbm.at[idxrow(sidx, i + 1)],
                                     rows.at[nb], gsem[nb])
                else:
                    @pl.when(t < _ECH // 2 - 1)
                    def _():
                        nxt = i + 1
                        blk = nxt // _IBLK

                        @pl.when(lax.rem(nxt, _IBLK) == 0)
                        def _():
                            @pl.when(lax.rem(blk, 2) == 0)
                            def _():
                                wait_iblk(0)

                            @pl.when(lax.rem(blk, 2) == 1)
                            def _():
                                wait_iblk(1)

                        pltpu.async_copy(xs_hbm.at[idxrow(sidx, nxt)],
                                         rows.at[nb], gsem[nb])
                pltpu.make_async_copy(
                    xs_hbm.at[sidx.at[0]], rows.at[b], gsem[b]).wait()
                pltpu.async_copy(rows.at[b], acc.at[idxrow(didx, i)],
                                 ssem[b], add=True)
                pltpu.make_async_copy(
                    rows.at[b], acc.at[didx.at[0]], ssem[b]).wait()
                if b == 1:
                    @pl.when(lax.rem(i, _IBLK) == _IBLK - 1)
                    def _():
                        blk2 = i // _IBLK + 2

                        @pl.when(blk2 < _NBLK)
                        def _():
                            @pl.when(lax.rem(blk2, 2) == 0)
                            def _():
                                fire_iblk(blk2, 0)

                            @pl.when(lax.rem(blk2, 2) == 1)
                            def _():
                                fire_iblk(blk2, 1)
            return carry

        lax.fori_loop(0, _ECH // 2, body, 0, unroll=False)
        plsc.subcore_barrier()
        _per_tile_slab(s, lambda b, n: pltpu.sync_copy(
            acc.at[pl.ds(b, n)], out_hbm.at[c, pl.ds(b, n)]))

    return k(xs_flat, src2d, dst2d)


def _sc_pair_gather(table, idx0r, idx1r):
    @functools.partial(
        pl.kernel,
        mesh=_sc_mesh(),
        out_type=(jax.ShapeDtypeStruct((_P2, _H), jnp.float32),
                  jax.ShapeDtypeStruct((_P2, _H), jnp.float32)),
        scratch_types=[
            pltpu.VMEM((_PCH, _PK), jnp.int32),
            pltpu.VMEM((_PCH, _PK), jnp.int32),
            pltpu.VMEM((3, _PK, _H), jnp.float32),
            pltpu.SemaphoreType.DMA,
            pltpu.SemaphoreType.DMA,
            pltpu.SemaphoreType.DMA,
            pltpu.SemaphoreType.DMA,
            pltpu.SemaphoreType.DMA,
            pltpu.SemaphoreType.DMA,
        ],
    )
    def k(tab_hbm, i0_hbm, i1_hbm, a0_hbm, a1_hbm, i0v, i1v, rows,
          g0, g1, g2, w0, w1, w2):
        c = lax.axis_index("c")
        s = lax.axis_index("s")
        w = s * _NC + c
        base = w * _PPW
        gsem = (g0, g1, g2)
        wsem = (w0, w1, w2)
        pltpu.sync_copy(i0_hbm.at[w], i0v)
        pltpu.sync_copy(i1_hbm.at[w], i1v)

        def stream(iv, out_hbm):
            for b in range(3):
                pltpu.async_copy(tab_hbm.at[iv.at[b]], rows.at[b], gsem[b])

            def body(j, carry):
                for b in range(3):
                    i = j * 3 + b
                    pltpu.make_async_copy(
                        tab_hbm.at[iv.at[0]], rows.at[b], gsem[b]).wait()
                    pltpu.async_copy(rows.at[b],
                                     out_hbm.at[pl.ds(base + i * _PK, _PK)],
                                     wsem[b])

                    @pl.when(i + 3 < _PCH)
                    def _():
                        pltpu.make_async_copy(
                            rows.at[b], out_hbm.at[pl.ds(base, _PK)],
                            wsem[b]).wait()
                        pltpu.async_copy(tab_hbm.at[iv.at[i + 3]], rows.at[b],
                                         gsem[b])
                return carry

            lax.fori_loop(0, (_PCH // 3) * 3 // 3, body, 0, unroll=False)
            pltpu.make_async_copy(
                tab_hbm.at[iv.at[0]], rows.at[0], gsem[0]).wait()
            pltpu.async_copy(rows.at[0],
                             out_hbm.at[pl.ds(base + (_PCH - 1) * _PK, _PK)],
                             wsem[0])
            for b in range(3):
                pltpu.make_async_copy(
                    rows.at[b], out_hbm.at[pl.ds(base, _PK)], wsem[b]).wait()

        stream(i0v, a0_hbm)
        stream(i1v, a1_hbm)

    return k(table, idx0r, idx1r)


_TB = 1000


def _dis_block(deg2):
    deg = deg2[0, :, 0:1] + deg2[1, :, 0:1] + 1.0
    return lax.rsqrt(deg)


def _c1_body(af, coords, deg2, we, be, gw1, gb1, gw2, gb2, w0, out):
    xe = jnp.dot(af[...], we[...], preferred_element_type=jnp.float32) + be[...]
    gh = jnp.maximum(
        jnp.dot(coords[...], gw1[...], preferred_element_type=jnp.float32) + gb1[...],
        0.0)
    gh = jnp.dot(gh, gw2[...], preferred_element_type=jnp.float32) + gb2[...]
    z = (jnp.dot(xe, w0[0:_H, :], preferred_element_type=jnp.float32)
         + jnp.dot(gh, w0[_H:, :], preferred_element_type=jnp.float32))
    xs = z * _dis_block(deg2[...])
    out[0, :, :] = xs[:, 0:_HH]
    out[1, :, :] = xs[:, _HH:]


def _tc_conv1(af, coords, deg2, we, be, gw1, gb1, gw2, gb2, w0):
    grid = _N // _TB
    return pl.pallas_call(
        _c1_body,
        grid=(grid,),
        in_specs=[
            pl.BlockSpec((_TB, 128), lambda i: (i, 0)),
            pl.BlockSpec((_TB, 3), lambda i: (i, 0)),
            pl.BlockSpec((_NC, _TB, _HH), lambda i: (0, i, 0)),
            pl.BlockSpec((128, _H), lambda i: (0, 0)),
            pl.BlockSpec((1, _H), lambda i: (0, 0)),
            pl.BlockSpec((3, 64), lambda i: (0, 0)),
            pl.BlockSpec((1, 64), lambda i: (0, 0)),
            pl.BlockSpec((64, 64), lambda i: (0, 0)),
            pl.BlockSpec((1, 64), lambda i: (0, 0)),
            pl.BlockSpec((_H + 64, _H), lambda i: (0, 0)),
        ],
        out_specs=pl.BlockSpec((_NC, _TB, _HH), lambda i: (0, i, 0)),
        out_shape=jax.ShapeDtypeStruct((_NC, _N, _HH), jnp.float32),
    )(af, coords, deg2, we, be, gw1, gb1, gw2, gb2, w0)


def _cmid_body(msg, deg2, b, gm, bt, w, out):
    dis = _dis_block(deg2[...])
    m = jnp.concatenate([msg[0, :, :], msg[1, :, :]], axis=1)
    x = jnp.maximum((m * dis + b[...]) * _BNS * gm[...] + bt[...], 0.0)
    xs = jnp.dot(x, w[...], preferred_element_type=jnp.float32) * dis
    out[0, :, :] = xs[:, 0:_HH]
    out[1, :, :] = xs[:, _HH:]


def _tc_conv_mid(msg, deg2, b, gm, bt, w):
    grid = _N // _TB
    return pl.pallas_call(
        _cmid_body,
        grid=(grid,),
        in_specs=[
            pl.BlockSpec((_NC, _TB, _HH), lambda i: (0, i, 0)),
            pl.BlockSpec((_NC, _TB, _HH), lambda i: (0, i, 0)),
            pl.BlockSpec((1, _H), lambda i: (0, 0)),
            pl.BlockSpec((1, _H), lambda i: (0, 0)),
            pl.BlockSpec((1, _H), lambda i: (0, 0)),
            pl.BlockSpec((_H, _H), lambda i: (0, 0)),
        ],
        out_specs=pl.BlockSpec((_NC, _TB, _HH), lambda i: (0, i, 0)),
        out_shape=jax.ShapeDtypeStruct((_NC, _N, _HH), jnp.float32),
    )(msg, deg2, b, gm, bt, w)


def _cfin_body(msg, deg2, b, gm, bt, out):
    dis = _dis_block(deg2[...])
    m = jnp.concatenate([msg[0, :, :], msg[1, :, :]], axis=1)
    out[...] = jnp.maximum((m * dis + b[...]) * _BNS * gm[...] + bt[...], 0.0)


def _tc_conv_fin(msg, deg2, b, gm, bt):
    grid = _N // _TB
    return pl.pallas_call(
        _cfin_body,
        grid=(grid,),
        in_specs=[
            pl.BlockSpec((_NC, _TB, _HH), lambda i: (0, i, 0)),
            pl.BlockSpec((_NC, _TB, _HH), lambda i: (0, i, 0)),
            pl.BlockSpec((1, _H), lambda i: (0, 0)),
            pl.BlockSpec((1, _H), lambda i: (0, 0)),
            pl.BlockSpec((1, _H), lambda i: (0, 0)),
        ],
        out_specs=pl.BlockSpec((_TB, _H), lambda i: (i, 0)),
        out_shape=jax.ShapeDtypeStruct((_N, _H), jnp.float32),
    )(msg, deg2, b, gm, bt)


_PB = 1024


def _pmlp_body(a0, a1, pfc, wa0, wa1, wf, b0, w1, b1, w2, b2, out):
    h = (jnp.dot(a0[...], wa0[...], preferred_element_type=jnp.float32)
         + jnp.dot(a1[...], wa1[...], preferred_element_type=jnp.float32)
         + jnp.dot(pfc[...], wf[...], preferred_element_type=jnp.float32)
         + b0[...])
    h = jnp.maximum(h, 0.0)
    h = jnp.maximum(jnp.dot(h, w1[...], preferred_element_type=jnp.float32) + b1[...], 0.0)
    out[...] = jnp.dot(h, w2[...], preferred_element_type=jnp.float32) + b2[...]


def _tc_pair_mlp(a0, a1, pfc, wa0, wa1, wf, b0, w1, b1, w2, b2):
    grid = _P2 // _PB
    h2 = 2 * _H
    return pl.pallas_call(
        _pmlp_body,
        grid=(grid,),
        in_specs=[
            pl.BlockSpec((_PB, _H), lambda i: (i, 0)),
            pl.BlockSpec((_PB, _H), lambda i: (i, 0)),
            pl.BlockSpec((_PB, 22), lambda i: (i, 0)),
            pl.BlockSpec((_H, h2), lambda i: (0, 0)),
            pl.BlockSpec((_H, h2), lambda i: (0, 0)),
            pl.BlockSpec((22, h2), lambda i: (0, 0)),
            pl.BlockSpec((1, h2), lambda i: (0, 0)),
            pl.BlockSpec((h2, _H), lambda i: (0, 0)),
            pl.BlockSpec((1, _H), lambda i: (0, 0)),
            pl.BlockSpec((_H, 1), lambda i: (0, 0)),
            pl.BlockSpec((1, 1), lambda i: (0, 0)),
        ],
        out_specs=pl.BlockSpec((_PB, 1), lambda i: (i, 0)),
        out_shape=jax.ShapeDtypeStruct((_P2, 1), jnp.float32),
    )(a0, a1, pfc, wa0, wa1, wf, b0, w1, b1, w2, b2)


def kernel(atom_features, atom_coords, edge_index, pair_indices, pair_features,
           pair_coords, W_embed, b_embed, gW1, gb1, gW2, gb2,
           conv_W0, conv_b0, conv_W1, conv_b1, conv_W2, conv_b2,
           bn_g0, bn_b0, bn_g1, bn_b1, bn_g2, bn_b2,
           pW0, pb0, pW1, pb1, pW2, pb2):
    src = edge_index[0].astype(jnp.int32)
    dst = edge_index[1].astype(jnp.int32)
    epad = _E2 - _E
    erange = jnp.arange(epad, dtype=jnp.int32)
    srcp = jnp.concatenate([src, (erange * 37) % _N])
    dstp = jnp.concatenate([dst, _N + (erange % _NTRASH)])
    src2d = jnp.concatenate([srcp, srcp + _N]).reshape(2 * _E2 // _PK, _PK)
    dst2d = dstp.reshape(_E2 // _PK, _PK)

    npad = _P2 - _P
    padidx = (jnp.arange(npad, dtype=jnp.int32) * 37) % _N
    idx0r = jnp.concatenate([pair_indices[:, 0].astype(jnp.int32), padidx]
                            ).reshape(_NC * _NS, _PCH, _PK)
    idx1r = jnp.concatenate([pair_indices[:, 1].astype(jnp.int32), padidx]
                            ).reshape(_NC * _NS, _PCH, _PK)
    pfc = jnp.concatenate(
        [pair_features, pair_coords.reshape(_P, 6)], axis=1)
    pfc = jnp.concatenate(
        [pfc, jnp.zeros((npad, 22), jnp.float32)], axis=0)

    zeros_nh = jnp.zeros((_N, _HH), jnp.float32)
    ones_kh = jnp.ones((_PK, _HH), jnp.float32)
    deg2 = _sc_degree(dst2d, zeros_nh, ones_kh)

    r = lambda v: v.reshape(1, -1)
    xs = _tc_conv1(atom_features, atom_coords, deg2, W_embed, r(b_embed),
                   gW1, r(gb1), gW2, r(gb2), conv_W0)
    msg = _sc_conv_scatter(xs.reshape(_NC * _N, _HH), src2d, dst2d)
    xs = _tc_conv_mid(msg, deg2, r(conv_b0), r(bn_g0), r(bn_b0), conv_W1)
    msg = _sc_conv_scatter(xs.reshape(_NC * _N, _HH), src2d, dst2d)
    xs = _tc_conv_mid(msg, deg2, r(conv_b1), r(bn_g1), r(bn_b1), conv_W2)
    msg = _sc_conv_scatter(xs.reshape(_NC * _N, _HH), src2d, dst2d)
    x3 = _tc_conv_fin(msg, deg2, r(conv_b2), r(bn_g2), r(bn_b2))

    a0g, a1g = _sc_pair_gather(x3, idx0r, idx1r)
    out = _tc_pair_mlp(a0g, a1g, pfc, pW0[0:_H], pW0[_H:2 * _H],
                       pW0[2 * _H:], r(pb0), pW1, r(pb1), pW2, r(pb2))
    return out[:_P]


# speedup vs baseline: 14.1546x; 1.0488x over previous
import functools

import jax
import jax.numpy as jnp
from jax import lax
from jax.experimental import pallas as pl
from jax.experimental.pallas import tpu as pltpu
from jax.experimental.pallas import tpu_sc as plsc

_N = 10000
_E = 320000
_P = 100000
_P2 = 102400
_H = 256
_HH = 128
_EPS = 1e-5
_BNS = float(1.0 / (1.0 + _EPS) ** 0.5)

_NC = 2
_NS = 16
_SLAB = 624
_PK = 128
_PPW = _P2 // (_NC * _NS)
_CNK = 5
_PPWC = _PPW // _CNK
_PCHC = _PPWC // _PK
_PC = _P2 // _CNK
_NTRASH = 16
_E2 = 327680
_ECH = _E2 // (_NS * _PK)
_DCH = _E2 // (_NC * _NS * _PK)
_IBLK = 8
_NBLK = _ECH // _IBLK


def _sc_mesh():
    return plsc.VectorSubcoreMesh(core_axis_name="c", subcore_axis_name="s")


def _per_tile_slab(s, do):
    @pl.when(s < _NS - 1)
    def _():
        do(s * _SLAB, _SLAB)

    @pl.when(s == _NS - 1)
    def _():
        do((_NS - 1) * _SLAB, _N - (_NS - 1) * _SLAB)


def _sc_degree(dst2d, zeros_nh, ones_kh):
    @functools.partial(
        pl.kernel,
        mesh=_sc_mesh(),
        out_type=jax.ShapeDtypeStruct((_NC, _N, _HH), jnp.float32),
        scratch_types=[
            pltpu.VMEM((_DCH, _PK), jnp.int32),
            pltpu.VMEM((_PK, _HH), jnp.float32),
            pltpu.VMEM_SHARED((_N + _NTRASH, _HH), jnp.float32),
            pltpu.SemaphoreType.DMA,
        ],
    )
    def k(dst_hbm, zero_hbm, ones_hbm, out_hbm, didx, ones_v, acc, sem):
        c = lax.axis_index("c")
        s = lax.axis_index("s")
        w = s * _NC + c
        pltpu.sync_copy(dst_hbm.at[pl.ds(w * _DCH, _DCH)], didx)
        pltpu.sync_copy(ones_hbm, ones_v)
        _per_tile_slab(s, lambda b, n: pltpu.sync_copy(
            zero_hbm.at[pl.ds(b, n)], acc.at[pl.ds(b, n)]))
        plsc.subcore_barrier()

        def body(j, carry):
            for b in range(8):
                pltpu.async_copy(ones_v, acc.at[didx.at[j * 8 + b]], sem,
                                 add=True)
            for b in range(8):
                pltpu.make_async_copy(ones_v, acc.at[didx.at[0]], sem).wait()
            return carry

        lax.fori_loop(0, _DCH // 8, body, 0, unroll=False)
        plsc.subcore_barrier()
        _per_tile_slab(s, lambda b, n: pltpu.sync_copy(
            acc.at[pl.ds(b, n)], out_hbm.at[c, pl.ds(b, n)]))

    return k(dst2d, zeros_nh, ones_kh)


def _sc_conv_scatter(xs_flat, src2d, dst2d):
    @functools.partial(
        pl.kernel,
        mesh=_sc_mesh(),
        out_type=jax.ShapeDtypeStruct((_NC, _N, _HH), jnp.float32),
        scratch_types=[
            pltpu.VMEM((2 * _IBLK, _PK), jnp.int32),
            pltpu.VMEM((2 * _IBLK, _PK), jnp.int32),
            pltpu.VMEM((2, _PK, _HH), jnp.float32),
            pltpu.VMEM_SHARED((_N + _NTRASH, _HH), jnp.float32),
            pltpu.SemaphoreType.DMA,
            pltpu.SemaphoreType.DMA,
            pltpu.SemaphoreType.DMA,
            pltpu.SemaphoreType.DMA,
            pltpu.SemaphoreType.DMA,
            pltpu.SemaphoreType.DMA,
        ],
    )
    def k(xs_hbm, src_hbm, dst_hbm, out_hbm, sidx, didx, rows, acc,
          g0, g1, s0, s1, i0, i1):
        c = lax.axis_index("c")
        s = lax.axis_index("s")
        sbase = (c * _NS + s) * _ECH
        dbase = s * _ECH
        gsem = (g0, g1)
        ssem = (s0, s1)
        isem = (i0, i1)

        def fire_iblk(blk, slot):
            pltpu.async_copy(src_hbm.at[pl.ds(sbase + blk * _IBLK, _IBLK)],
                             sidx.at[pl.ds(slot * _IBLK, _IBLK)], isem[slot])
            pltpu.async_copy(dst_hbm.at[pl.ds(dbase + blk * _IBLK, _IBLK)],
                             didx.at[pl.ds(slot * _IBLK, _IBLK)], isem[slot])

        def wait_iblk(slot):
            for _ in range(2):
                pltpu.make_async_copy(
                    src_hbm.at[pl.ds(0, _IBLK)],
                    sidx.at[pl.ds(0, _IBLK)], isem[slot]).wait()

        def idxrow(ref, i):
            slot = lax.rem(i // _IBLK, 2)
            return ref.at[slot * _IBLK + lax.rem(i, _IBLK)]

        fire_iblk(0, 0)
        fire_iblk(1, 1)
        _per_tile_slab(s, lambda b, n: pltpu.sync_copy(
            xs_hbm.at[pl.ds(c * _N + b, n)], acc.at[pl.ds(b, n)]))
        plsc.subcore_barrier()
        wait_iblk(0)
        pltpu.async_copy(xs_hbm.at[sidx.at[0]], rows.at[0], g0)

        def body(t, carry):
            for b in (0, 1):
                i = t * 2 + b
                nb = 1 - b
                if b == 0:
                    pltpu.async_copy(xs_hbm.at[idxrow(sidx, i + 1)],
                                     rows.at[nb], gsem[nb])
                else:
                    @pl.when(t < _ECH // 2 - 1)
                    def _():
                        nxt = i + 1
                        blk = nxt // _IBLK

                        @pl.when(lax.rem(nxt, _IBLK) == 0)
                        def _():
                            @pl.when(lax.rem(blk, 2) == 0)
                            def _():
                                wait_iblk(0)

                            @pl.when(lax.rem(blk, 2) == 1)
                            def _():
                                wait_iblk(1)

                        pltpu.async_copy(xs_hbm.at[idxrow(sidx, nxt)],
                                         rows.at[nb], gsem[nb])
                pltpu.make_async_copy(
                    xs_hbm.at[sidx.at[0]], rows.at[b], gsem[b]).wait()
                pltpu.async_copy(rows.at[b], acc.at[idxrow(didx, i)],
                                 ssem[b], add=True)
                pltpu.make_async_copy(
                    rows.at[b], acc.at[didx.at[0]], ssem[b]).wait()
                if b == 1:
                    @pl.when(lax.rem(i, _IBLK) == _IBLK - 1)
                    def _():
                        blk2 = i // _IBLK + 2

                        @pl.when(blk2 < _NBLK)
                        def _():
                            @pl.when(lax.rem(blk2, 2) == 0)
                            def _():
                                fire_iblk(blk2, 0)

                            @pl.when(lax.rem(blk2, 2) == 1)
                            def _():
                                fire_iblk(blk2, 1)
            return carry

        lax.fori_loop(0, _ECH // 2, body, 0, unroll=False)
        plsc.subcore_barrier()
        _per_tile_slab(s, lambda b, n: pltpu.sync_copy(
            acc.at[pl.ds(b, n)], out_hbm.at[c, pl.ds(b, n)]))

    return k(xs_flat, src2d, dst2d)


def _sc_pair_gather(table, idx0c, idx1c):
    @functools.partial(
        pl.kernel,
        mesh=_sc_mesh(),
        out_type=(jax.ShapeDtypeStruct((_PC, _H), jnp.float32),
                  jax.ShapeDtypeStruct((_PC, _H), jnp.float32)),
        scratch_types=[
            pltpu.VMEM((_PCHC, _PK), jnp.int32),
            pltpu.VMEM((_PCHC, _PK), jnp.int32),
            pltpu.VMEM((3, _PK, _H), jnp.float32),
            pltpu.SemaphoreType.DMA,
            pltpu.SemaphoreType.DMA,
            pltpu.SemaphoreType.DMA,
            pltpu.SemaphoreType.DMA,
            pltpu.SemaphoreType.DMA,
            pltpu.SemaphoreType.DMA,
        ],
    )
    def k(tab_hbm, i0_hbm, i1_hbm, a0_hbm, a1_hbm, i0v, i1v, rows,
          g0, g1, g2, w0, w1, w2):
        c = lax.axis_index("c")
        s = lax.axis_index("s")
        w = s * _NC + c
        base = w * _PPWC
        gsem = (g0, g1, g2)
        wsem = (w0, w1, w2)
        pltpu.sync_copy(i0_hbm.at[w], i0v)
        pltpu.sync_copy(i1_hbm.at[w], i1v)

        def stream(iv, out_hbm):
            for b in range(3):
                pltpu.async_copy(tab_hbm.at[iv.at[b]], rows.at[b], gsem[b])
            for i in range(_PCHC):
                b = i % 3
                pltpu.make_async_copy(
                    tab_hbm.at[iv.at[0]], rows.at[b], gsem[b]).wait()
                pltpu.async_copy(rows.at[b],
                                 out_hbm.at[pl.ds(base + i * _PK, _PK)],
                                 wsem[b])
                if i + 3 < _PCHC:
                    pltpu.make_async_copy(
                        rows.at[b], out_hbm.at[pl.ds(base, _PK)],
                        wsem[b]).wait()
                    pltpu.async_copy(tab_hbm.at[iv.at[i + 3]], rows.at[b],
                                     gsem[b])
            for b in range(3):
                pltpu.make_async_copy(
                    rows.at[b], out_hbm.at[pl.ds(base, _PK)], wsem[b]).wait()

        stream(i0v, a0_hbm)
        stream(i1v, a1_hbm)

    return k(table, idx0c, idx1c)


_TB = 1000


def _dis_block(deg2):
    deg = deg2[0, :, 0:1] + deg2[1, :, 0:1] + 1.0
    return lax.rsqrt(deg)


def _c1_body(af, coords, we, be, gw1, gb1, gw2, gb2, w0, out):
    xe = jnp.dot(af[...], we[...], preferred_element_type=jnp.float32) + be[...]
    gh = jnp.maximum(
        jnp.dot(coords[...], gw1[...], preferred_element_type=jnp.float32) + gb1[...],
        0.0)
    gh = jnp.dot(gh, gw2[...], preferred_element_type=jnp.float32) + gb2[...]
    z = (jnp.dot(xe, w0[0:_H, :], preferred_element_type=jnp.float32)
         + jnp.dot(gh, w0[_H:, :], preferred_element_type=jnp.float32))
    out[0, :, :] = z[:, 0:_HH]
    out[1, :, :] = z[:, _HH:]


def _tc_conv1_mm(af, coords, we, be, gw1, gb1, gw2, gb2, w0):
    grid = _N // _TB
    return pl.pallas_call(
        _c1_body,
        grid=(grid,),
        in_specs=[
            pl.BlockSpec((_TB, 128), lambda i: (i, 0)),
            pl.BlockSpec((_TB, 3), lambda i: (i, 0)),
            pl.BlockSpec((128, _H), lambda i: (0, 0)),
            pl.BlockSpec((1, _H), lambda i: (0, 0)),
            pl.BlockSpec((3, 64), lambda i: (0, 0)),
            pl.BlockSpec((1, 64), lambda i: (0, 0)),
            pl.BlockSpec((64, 64), lambda i: (0, 0)),
            pl.BlockSpec((1, 64), lambda i: (0, 0)),
            pl.BlockSpec((_H + 64, _H), lambda i: (0, 0)),
        ],
        out_specs=pl.BlockSpec((_NC, _TB, _HH), lambda i: (0, i, 0)),
        out_shape=jax.ShapeDtypeStruct((_NC, _N, _HH), jnp.float32),
    )(af, coords, we, be, gw1, gb1, gw2, gb2, w0)


def _scale_body(z, deg2, out):
    dis = _dis_block(deg2[...])
    out[0, :, :] = z[0, :, :] * dis
    out[1, :, :] = z[1, :, :] * dis


def _tc_scale(z, deg2):
    grid = _N // _TB
    return pl.pallas_call(
        _scale_body,
        grid=(grid,),
        in_specs=[
            pl.BlockSpec((_NC, _TB, _HH), lambda i: (0, i, 0)),
            pl.BlockSpec((_NC, _TB, _HH), lambda i: (0, i, 0)),
        ],
        out_specs=pl.BlockSpec((_NC, _TB, _HH), lambda i: (0, i, 0)),
        out_shape=jax.ShapeDtypeStruct((_NC, _N, _HH), jnp.float32),
    )(z, deg2)


def _cmid_body(msg, deg2, b, gm, bt, w, out):
    dis = _dis_block(deg2[...])
    m = jnp.concatenate([msg[0, :, :], msg[1, :, :]], axis=1)
    x = jnp.maximum((m * dis + b[...]) * _BNS * gm[...] + bt[...], 0.0)
    xs = jnp.dot(x, w[...], preferred_element_type=jnp.float32) * dis
    out[0, :, :] = xs[:, 0:_HH]
    out[1, :, :] = xs[:, _HH:]


def _tc_conv_mid(msg, deg2, b, gm, bt, w):
    grid = _N // _TB
    return pl.pallas_call(
        _cmid_body,
        grid=(grid,),
        in_specs=[
            pl.BlockSpec((_NC, _TB, _HH), lambda i: (0, i, 0)),
            pl.BlockSpec((_NC, _TB, _HH), lambda i: (0, i, 0)),
            pl.BlockSpec((1, _H), lambda i: (0, 0)),
            pl.BlockSpec((1, _H), lambda i: (0, 0)),
            pl.BlockSpec((1, _H), lambda i: (0, 0)),
            pl.BlockSpec((_H, _H), lambda i: (0, 0)),
        ],
        out_specs=pl.BlockSpec((_NC, _TB, _HH), lambda i: (0, i, 0)),
        out_shape=jax.ShapeDtypeStruct((_NC, _N, _HH), jnp.float32),
    )(msg, deg2, b, gm, bt, w)


def _cfin_body(msg, deg2, b, gm, bt, out):
    dis = _dis_block(deg2[...])
    m = jnp.concatenate([msg[0, :, :], msg[1, :, :]], axis=1)
    out[...] = jnp.maximum((m * dis + b[...]) * _BNS * gm[...] + bt[...], 0.0)


def _tc_conv_fin(msg, deg2, b, gm, bt):
    grid = _N // _TB
    return pl.pallas_call(
        _cfin_body,
        grid=(grid,),
        in_specs=[
            pl.BlockSpec((_NC, _TB, _HH), lambda i: (0, i, 0)),
            pl.BlockSpec((_NC, _TB, _HH), lambda i: (0, i, 0)),
            pl.BlockSpec((1, _H), lambda i: (0, 0)),
            pl.BlockSpec((1, _H), lambda i: (0, 0)),
            pl.BlockSpec((1, _H), lambda i: (0, 0)),
        ],
        out_specs=pl.BlockSpec((_TB, _H), lambda i: (i, 0)),
        out_shape=jax.ShapeDtypeStruct((_N, _H), jnp.float32),
    )(msg, deg2, b, gm, bt)


_PB = 1024


def _pmlp_body(a0, a1, pfc, wa0, wa1, wf, b0, w1, b1, w2, b2, out):
    h = (jnp.dot(a0[...], wa0[...], preferred_element_type=jnp.float32)
         + jnp.dot(a1[...], wa1[...], preferred_element_type=jnp.float32)
         + jnp.dot(pfc[...], wf[...], preferred_element_type=jnp.float32)
         + b0[...])
    h = jnp.maximum(h, 0.0)
    h = jnp.maximum(jnp.dot(h, w1[...], preferred_element_type=jnp.float32) + b1[...], 0.0)
    out[...] = jnp.dot(h, w2[...], preferred_element_type=jnp.float32) + b2[...]


def _tc_pair_mlp(a0, a1, pfc, wa0, wa1, wf, b0, w1, b1, w2, b2):
    grid = _PC // _PB
    h2 = 2 * _H
    return pl.pallas_call(
        _pmlp_body,
        grid=(grid,),
        in_specs=[
            pl.BlockSpec((_PB, _H), lambda i: (i, 0)),
            pl.BlockSpec((_PB, _H), lambda i: (i, 0)),
            pl.BlockSpec((_PB, 22), lambda i: (i, 0)),
            pl.BlockSpec((_H, h2), lambda i: (0, 0)),
            pl.BlockSpec((_H, h2), lambda i: (0, 0)),
            pl.BlockSpec((22, h2), lambda i: (0, 0)),
            pl.BlockSpec((1, h2), lambda i: (0, 0)),
            pl.BlockSpec((h2, _H), lambda i: (0, 0)),
            pl.BlockSpec((1, _H), lambda i: (0, 0)),
            pl.BlockSpec((_H, 1), lambda i: (0, 0)),
            pl.BlockSpec((1, 1), lambda i: (0, 0)),
        ],
        out_specs=pl.BlockSpec((_PB, 1), lambda i: (i, 0)),
        out_shape=jax.ShapeDtypeStruct((_PC, 1), jnp.float32),
    )(a0, a1, pfc, wa0, wa1, wf, b0, w1, b1, w2, b2)


def kernel(atom_features, atom_coords, edge_index, pair_indices, pair_features,
           pair_coords, W_embed, b_embed, gW1, gb1, gW2, gb2,
           conv_W0, conv_b0, conv_W1, conv_b1, conv_W2, conv_b2,
           bn_g0, bn_b0, bn_g1, bn_b1, bn_g2, bn_b2,
           pW0, pb0, pW1, pb1, pW2, pb2):
    src = edge_index[0].astype(jnp.int32)
    dst = edge_index[1].astype(jnp.int32)
    epad = _E2 - _E
    erange = jnp.arange(epad, dtype=jnp.int32)
    srcp = jnp.concatenate([src, (erange * 37) % _N])
    dstp = jnp.concatenate([dst, _N + (erange % _NTRASH)])
    src2d = jnp.concatenate([srcp, srcp + _N]).reshape(2 * _E2 // _PK, _PK)
    dst2d = dstp.reshape(_E2 // _PK, _PK)

    npad = _P2 - _P
    padidx = (jnp.arange(npad, dtype=jnp.int32) * 37) % _N
    chunked = lambda v: (v.reshape(_NC * _NS, _CNK, _PPWC)
                         .transpose(1, 0, 2)
                         .reshape(_CNK, _NC * _NS, _PCHC, _PK))
    idx0r = chunked(jnp.concatenate(
        [pair_indices[:, 0].astype(jnp.int32), padidx]))
    idx1r = chunked(jnp.concatenate(
        [pair_indices[:, 1].astype(jnp.int32), padidx]))
    pfc = jnp.concatenate(
        [pair_features, pair_coords.reshape(_P, 6)], axis=1)
    pfc = jnp.concatenate(
        [pfc, jnp.zeros((npad, 22), jnp.float32)], axis=0)
    pfc = (pfc.reshape(_NC * _NS, _CNK, _PPWC, 22)
           .transpose(1, 0, 2, 3).reshape(_CNK, _PC, 22))

    zeros_nh = jnp.zeros((_N, _HH), jnp.float32)
    ones_kh = jnp.ones((_PK, _HH), jnp.float32)

    r = lambda v: v.reshape(1, -1)
    deg2 = _sc_degree(dst2d, zeros_nh, ones_kh)
    z = _tc_conv1_mm(atom_features, atom_coords, W_embed, r(b_embed),
                     gW1, r(gb1), gW2, r(gb2), conv_W0)
    xs = _tc_scale(z, deg2)
    msg = _sc_conv_scatter(xs.reshape(_NC * _N, _HH), src2d, dst2d)
    xs = _tc_conv_mid(msg, deg2, r(conv_b0), r(bn_g0), r(bn_b0), conv_W1)
    msg = _sc_conv_scatter(xs.reshape(_NC * _N, _HH), src2d, dst2d)
    xs = _tc_conv_mid(msg, deg2, r(conv_b1), r(bn_g1), r(bn_b1), conv_W2)
    msg = _sc_conv_scatter(xs.reshape(_NC * _N, _HH), src2d, dst2d)
    x3 = _tc_conv_fin(msg, deg2, r(conv_b2), r(bn_g2), r(bn_b2))

    outs = []
    for k in range(_CNK):
        a0g, a1g = _sc_pair_gather(x3, idx0r[k], idx1r[k])
        outs.append(_tc_pair_mlp(
            a0g, a1g, pfc[k], pW0[0:_H], pW0[_H:2 * _H],
            pW0[2 * _H:], r(pb0), pW1, r(pb1), pW2, r(pb2)))
    out = (jnp.stack(outs).reshape(_CNK, _NC * _NS, _PPWC)
           .transpose(1, 0, 2).reshape(_P2, 1))
    return out[:_P]
